# baseline (speedup 1.0000x reference)
import functools

import jax
import jax.numpy as jnp
from jax.experimental import pallas as pl
from jax.experimental.pallas import tpu as pltpu

NUM_LABELED = 15080
OUT_CHANNELS = 2048
TEMP = 0.05
N_ROWS = 64

NSTREAM = 4
HALF = 472
BLOCK = NSTREAM * HALF
NB = (NUM_LABELED + BLOCK - 1) // BLOCK
assert (NSTREAM * (NB - 1) + NSTREAM - 1) * HALF < NUM_LABELED


def _loss_kernel(feat_ref, pids_ref, *refs):
    mem_refs = refs[:NSTREAM]
    out_ref, m_ref, s_ref, p_ref = refs[NSTREAM:]
    k = pl.program_id(0)

    pids = pids_ref[...][:, 4:5]
    mask = pids > -1
    targets = jnp.where(mask, pids, 0)

    feat = feat_ref[...]
    feat = jnp.where(mask, feat, 0.0)

    dn = (((1,), (1,)), ((), ()))
    parts = [
        jax.lax.dot_general(feat, r[...], dimension_numbers=dn,
                            preferred_element_type=jnp.float32)
        for r in mem_refs
    ]
    p = jnp.concatenate(parts, axis=1) * (1.0 / TEMP)

    col = k * BLOCK + jax.lax.broadcasted_iota(jnp.int32, (N_ROWS, BLOCK), 1)
    valid = col < NUM_LABELED
    neg = jnp.float32(-jnp.inf)
    pv = jnp.where(valid, p, neg)

    hit = col == targets
    p_blk = jnp.sum(jnp.where(hit, p, 0.0), axis=1, keepdims=True)

    @pl.when(k == 0)
    def _init():
        m_ref[...] = jnp.full((N_ROWS, 1), neg, jnp.float32)
        s_ref[...] = jnp.zeros((N_ROWS, 1), jnp.float32)
        p_ref[...] = jnp.zeros((N_ROWS, 1), jnp.float32)

    m_prev = m_ref[...]
    s_prev = s_ref[...]
    bmax = jnp.max(pv, axis=1, keepdims=True)
    m_new = jnp.maximum(m_prev, bmax)
    s_new = s_prev * jnp.exp(m_prev - m_new) + jnp.sum(
        jnp.exp(pv - m_new), axis=1, keepdims=True)
    m_ref[...] = m_new
    s_ref[...] = s_new
    p_ref[...] = p_ref[...] + p_blk

    @pl.when(k == NB - 1)
    def _finish():
        lse = m_new + jnp.log(s_new)
        maskf = mask.astype(jnp.float32)
        picked = p_ref[...]
        loss = -jnp.sum((picked - lse) * maskf) / jnp.sum(maskf)
        out_ref[0, 0] = loss


@jax.jit
def _run(feat, pids2d, memory):
    out = pl.pallas_call(
        _loss_kernel,
        grid=(NB,),
        in_specs=[
            pl.BlockSpec((N_ROWS, OUT_CHANNELS), lambda k: (0, 0)),
            pl.BlockSpec((N_ROWS, 5), lambda k: (0, 0)),
        ] + [
            pl.BlockSpec((HALF, OUT_CHANNELS),
                         functools.partial(lambda q, k: (NSTREAM * k + q, 0),
                                           q))
            for q in range(NSTREAM)
        ],
        out_specs=pl.BlockSpec(memory_space=pltpu.SMEM),
        out_shape=jax.ShapeDtypeStruct((1, 1), jnp.float32),
        scratch_shapes=[
            pltpu.VMEM((N_ROWS, 1), jnp.float32),
            pltpu.VMEM((N_ROWS, 1), jnp.float32),
            pltpu.VMEM((N_ROWS, 1), jnp.float32),
        ],
        compiler_params=pltpu.CompilerParams(
            dimension_semantics=("arbitrary",),
        ),
    )(feat, pids2d, *([memory] * NSTREAM))
    return out[0, 0]


def kernel(features, gt_labels, memory):
    boxes = gt_labels.reshape(N_ROWS, 5).astype(jnp.int32)
    return _run(features, boxes, memory)
